# baseline (speedup 1.0000x reference)
import functools

import jax
import jax.numpy as jnp
from jax import lax
from jax.experimental import pallas as pl
from jax.experimental.pallas import tpu as pltpu
from jax.experimental.pallas import tpu_sc as plsc

_NC = 2
_NS = 16
_NW = _NC * _NS
_CB = 128


@functools.lru_cache(maxsize=None)
def _make_gather(V, D, B):
    assert B % (_NW * _CB) == 0
    bpw = B // _NW
    nch = bpw // _CB
    mesh = plsc.VectorSubcoreMesh(
        core_axis_name="c", subcore_axis_name="s",
        num_cores=_NC, num_subcores=_NS,
    )

    @functools.partial(
        pl.kernel,
        out_type=jax.ShapeDtypeStruct((B, D), jnp.float32),
        mesh=mesh,
        scratch_types=[
            pltpu.VMEM((nch, _CB), jnp.int32),
            pltpu.VMEM((_CB, D), jnp.float32),
            pltpu.SemaphoreType.DMA,
        ],
        compiler_params=pltpu.CompilerParams(use_tc_tiling_on_sc=False),
    )
    def k(idx_hbm, table_hbm, out_hbm, idx_v, buf, sem):
        wid = lax.axis_index("s") * _NC + lax.axis_index("c")
        base = wid * bpw
        pltpu.sync_copy(idx_hbm.at[wid], idx_v)

        @pl.loop(0, nch)
        def _(j):
            pltpu.async_copy(table_hbm.at[idx_v.at[j]], buf, sem).wait()
            pltpu.sync_copy(buf, out_hbm.at[pl.ds(base + j * _CB, _CB)])

    return k


def kernel(word_seqs, table):
    Bm, S = word_seqs.shape
    V, D = table.shape
    B = Bm * S
    idx = word_seqs.reshape(_NW, B // (_NW * _CB), _CB).astype(jnp.int32)
    out = _make_gather(V, D, B)(idx, table)
    return out.reshape(Bm, S, D)


# speedup vs baseline: 1.0438x; 1.0438x over previous
import functools

import jax
import jax.numpy as jnp
from jax import lax
from jax.experimental import pallas as pl
from jax.experimental.pallas import tpu as pltpu
from jax.experimental.pallas import tpu_sc as plsc

_NC = 2
_NS = 16
_NW = _NC * _NS
_CB = 1600


@functools.lru_cache(maxsize=None)
def _make_gather(V, D, B):
    assert B % (_NW * _CB) == 0
    bpw = B // _NW
    nch = bpw // _CB
    mesh = plsc.VectorSubcoreMesh(
        core_axis_name="c", subcore_axis_name="s",
        num_cores=_NC, num_subcores=_NS,
    )

    @functools.partial(
        pl.kernel,
        out_type=jax.ShapeDtypeStruct((B, D), jnp.float32),
        mesh=mesh,
        scratch_types=[
            pltpu.VMEM((nch, _CB), jnp.int32),
            pltpu.VMEM((_CB, D), jnp.float32),
            pltpu.SemaphoreType.DMA,
        ],
        compiler_params=pltpu.CompilerParams(use_tc_tiling_on_sc=False),
    )
    def k(idx_hbm, table_hbm, out_hbm, idx_v, buf, sem):
        wid = lax.axis_index("s") * _NC + lax.axis_index("c")
        base = wid * bpw
        pltpu.sync_copy(idx_hbm.at[wid], idx_v)

        @pl.loop(0, nch)
        def _(j):
            pltpu.async_copy(table_hbm.at[idx_v.at[j]], buf, sem).wait()
            pltpu.sync_copy(buf, out_hbm.at[pl.ds(base + j * _CB, _CB)])

    return k


def kernel(word_seqs, table):
    Bm, S = word_seqs.shape
    V, D = table.shape
    B = Bm * S
    idx = word_seqs.reshape(_NW, B // (_NW * _CB), _CB).astype(jnp.int32)
    out = _make_gather(V, D, B)(idx, table)
    return out.reshape(Bm, S, D)


# speedup vs baseline: 1.1085x; 1.0620x over previous
import functools

import jax
import jax.numpy as jnp
from jax import lax
from jax.experimental import pallas as pl
from jax.experimental.pallas import tpu as pltpu
from jax.experimental.pallas import tpu_sc as plsc

_NC = 2
_NS = 16
_NW = _NC * _NS
_CB = 128
_L = 16


@functools.lru_cache(maxsize=None)
def _make_gather(V, D, Bm, S):
    assert Bm % (_NW * _CB) == 0 or Bm == _NW * _CB
    btiles = Bm // _CB
    assert btiles == _NW, "one batch tile per subcore"
    mesh = plsc.VectorSubcoreMesh(
        core_axis_name="c", subcore_axis_name="s",
        num_cores=_NC, num_subcores=_NS,
    )

    @functools.partial(
        pl.kernel,
        out_type=jax.ShapeDtypeStruct((S, D, Bm), jnp.float32),
        mesh=mesh,
        scratch_types=[
            pltpu.VMEM((S, _CB), jnp.int32),
            pltpu.VMEM((_CB, D), jnp.float32),
            pltpu.VMEM((D, _CB), jnp.float32),
            pltpu.SemaphoreType.DMA,
        ],
        compiler_params=pltpu.CompilerParams(
            use_tc_tiling_on_sc=False, needs_layout_passes=False),
    )
    def k(ws_hbm, table_hbm, out_hbm, idx_v, buf, buf_t, sem):
        w = lax.axis_index("s") * _NC + lax.axis_index("c")
        col = w * _CB
        pltpu.sync_copy(ws_hbm.at[:, pl.ds(col, _CB)], idx_v)

        rows = [lax.iota(jnp.int32, _L) + kk * _L for kk in range(_CB // _L)]

        @pl.loop(0, S)
        def _(s):
            pltpu.async_copy(table_hbm.at[idx_v.at[s]], buf, sem).wait()
            for d in range(D):
                dcol = jnp.full((_L,), d, jnp.int32)
                for kk in range(_CB // _L):
                    v = plsc.load_gather(buf, [rows[kk], dcol])
                    buf_t[d, pl.ds(kk * _L, _L)] = v
            pltpu.sync_copy(buf_t, out_hbm.at[s, :, pl.ds(col, _CB)])

    return k


def kernel(word_seqs, table):
    Bm, S = word_seqs.shape
    V, D = table.shape
    ws_t = word_seqs.T.astype(jnp.int32)
    out_t = _make_gather(V, D, Bm, S)(ws_t, table)
    return out_t.transpose(2, 0, 1)


# speedup vs baseline: 1.1577x; 1.0443x over previous
import functools

import jax
import jax.numpy as jnp
from jax import lax
from jax.experimental import pallas as pl
from jax.experimental.pallas import tpu as pltpu
from jax.experimental.pallas import tpu_sc as plsc

_NC = 2
_NS = 16
_NW = _NC * _NS
_CB = 128
_L = 16


@functools.lru_cache(maxsize=None)
def _make_gather(V, D, Bm, S):
    assert Bm % (_NW * _CB) == 0 or Bm == _NW * _CB
    btiles = Bm // _CB
    assert btiles == _NW, "one batch tile per subcore"
    mesh = plsc.VectorSubcoreMesh(
        core_axis_name="c", subcore_axis_name="s",
        num_cores=_NC, num_subcores=_NS,
    )

    @functools.partial(
        pl.kernel,
        out_type=jax.ShapeDtypeStruct((S, D, Bm), jnp.float32),
        mesh=mesh,
        scratch_types=[
            pltpu.VMEM((S, _CB), jnp.int32),
            pltpu.VMEM((_CB, D), jnp.float32),
            pltpu.VMEM((D, _CB), jnp.float32),
            pltpu.SemaphoreType.DMA,
        ],
        compiler_params=pltpu.CompilerParams(
            use_tc_tiling_on_sc=False, needs_layout_passes=False),
    )
    def k(ws_hbm, table_hbm, out_hbm, idx_v, buf, buf_t, sem):
        w = lax.axis_index("s") * _NC + lax.axis_index("c")
        col = w * _CB
        pltpu.sync_copy(ws_hbm.at[:, pl.ds(col, _CB)], idx_v)

        rows = [lax.iota(jnp.int32, _L) + kk * _L for kk in range(_CB // _L)]

        @pl.loop(0, S)
        def _(s):
            pltpu.async_copy(table_hbm.at[idx_v.at[s]], buf, sem).wait()
            pairs = [(d, kk) for d in range(D) for kk in range(_CB // _L)]
            for i in range(0, len(pairs), 8):
                batch = pairs[i:i + 8]
                vs = [
                    plsc.load_gather(buf, [rows[kk], jnp.full((_L,), d, jnp.int32)])
                    for d, kk in batch
                ]
                for (d, kk), v in zip(batch, vs):
                    buf_t[d, pl.ds(kk * _L, _L)] = v
            pltpu.sync_copy(buf_t, out_hbm.at[s, :, pl.ds(col, _CB)])

    return k


def kernel(word_seqs, table):
    Bm, S = word_seqs.shape
    V, D = table.shape
    ws_t = word_seqs.T.astype(jnp.int32)
    out_t = _make_gather(V, D, Bm, S)(ws_t, table)
    return out_t.transpose(2, 0, 1)
